# baseline (speedup 1.0000x reference)
import functools

import jax
import jax.numpy as jnp
from jax import lax
from jax.experimental import pallas as pl
from jax.experimental.pallas import tpu as pltpu
from jax.experimental.pallas import tpu_sc as plsc

BATCH = 16384
EMBED_DIM = 16
NC = 2
NS = 16
NW = NC * NS
B_PER_W = BATCH // NW


def _sc_gather(user_ids, item_ids, cf_user_table, cf_item_table):
    mesh = plsc.VectorSubcoreMesh(core_axis_name="c", subcore_axis_name="s")

    @functools.partial(
        pl.kernel,
        mesh=mesh,
        out_type=[
            jax.ShapeDtypeStruct((BATCH, EMBED_DIM), jnp.float32),
            jax.ShapeDtypeStruct((BATCH, EMBED_DIM), jnp.float32),
        ],
        scratch_types=[
            pltpu.VMEM((B_PER_W,), jnp.int32),
            pltpu.VMEM((B_PER_W, EMBED_DIM), jnp.float32),
            pltpu.VMEM((B_PER_W,), jnp.int32),
            pltpu.VMEM((B_PER_W, EMBED_DIM), jnp.float32),
            pltpu.SemaphoreType.DMA,
            pltpu.SemaphoreType.DMA,
        ],
        compiler_params=pltpu.CompilerParams(use_tc_tiling_on_sc=False),
    )
    def gather_kernel(uid_hbm, iid_hbm, ut_hbm, it_hbm, out_u_hbm, out_i_hbm,
                      idx_u, rows_u, idx_i, rows_i, sem_u, sem_i):
        wid = lax.axis_index("s") * NC + lax.axis_index("c")
        base = wid * B_PER_W
        pltpu.sync_copy(uid_hbm.at[pl.ds(base, B_PER_W)], idx_u)
        pltpu.sync_copy(iid_hbm.at[pl.ds(base, B_PER_W)], idx_i)
        cp_u = pltpu.async_copy(ut_hbm.at[idx_u], rows_u, sem_u)
        cp_i = pltpu.async_copy(it_hbm.at[idx_i], rows_i, sem_i)
        cp_u.wait()
        cp_i.wait()
        pltpu.sync_copy(rows_u, out_u_hbm.at[pl.ds(base, B_PER_W)])
        pltpu.sync_copy(rows_i, out_i_hbm.at[pl.ds(base, B_PER_W)])

    return gather_kernel(user_ids, item_ids, cf_user_table, cf_item_table)


def _tc_combine_body(uf_ref, cfu_ref, cfi_ref, w1_ref, b1_ref, w2_ref, b2_ref,
                     out_ref):
    h = jnp.maximum(
        jnp.dot(uf_ref[...], w1_ref[...], preferred_element_type=jnp.float32)
        + b1_ref[...], 0.0)
    content = jnp.dot(h, w2_ref[...], preferred_element_type=jnp.float32) \
        + b2_ref[...]
    out_ref[...] = jnp.sum((cfu_ref[...] + content) * cfi_ref[...], axis=1)


def _tc_combine(user_features, cf_u, cf_i, W1, b1, W2, b2):
    return pl.pallas_call(
        _tc_combine_body,
        out_shape=jax.ShapeDtypeStruct((BATCH,), jnp.float32),
    )(user_features, cf_u, cf_i, W1, b1.reshape(1, 32), W2,
      b2.reshape(1, EMBED_DIM))


def kernel(user_ids, item_ids, user_features, cf_user_table, cf_item_table,
           W1, b1, W2, b2):
    cf_u, cf_i = _sc_gather(user_ids, item_ids, cf_user_table, cf_item_table)
    return _tc_combine(user_features, cf_u, cf_i, W1, b1, W2, b2)


# speedup vs baseline: 4.9322x; 4.9322x over previous
import functools

import jax
import jax.numpy as jnp
from jax import lax
from jax.experimental import pallas as pl
from jax.experimental.pallas import tpu as pltpu
from jax.experimental.pallas import tpu_sc as plsc

BATCH = 16384
EMBED_DIM = 16
N_ROWS = 1000000
TAIL_START = (N_ROWS // 128) * 128
NC = 2
NS = 16
NW = NC * NS
B_PER_W = BATCH // NW
GRP = 16
C_CHUNK = 128


def _tc_mlp_body(uf_ref, w1_ref, b1_ref, w2_ref, b2_ref, out_ref):
    h = jnp.maximum(
        jnp.dot(uf_ref[...], w1_ref[...], preferred_element_type=jnp.float32)
        + b1_ref[...], 0.0)
    c = jnp.dot(h, w2_ref[...], preferred_element_type=jnp.float32) \
        + b2_ref[...]
    out_ref[...] = jnp.pad(c, ((0, 0), (0, 128 - EMBED_DIM)))


def _tc_mlp(user_features, W1, b1, W2, b2):
    blk = 2048
    return pl.pallas_call(
        _tc_mlp_body,
        grid=(BATCH // blk,),
        in_specs=[
            pl.BlockSpec((blk, 64), lambda i: (i, 0)),
            pl.BlockSpec((64, 32), lambda i: (0, 0)),
            pl.BlockSpec((1, 32), lambda i: (0, 0)),
            pl.BlockSpec((32, EMBED_DIM), lambda i: (0, 0)),
            pl.BlockSpec((1, EMBED_DIM), lambda i: (0, 0)),
        ],
        out_specs=pl.BlockSpec((blk, 128), lambda i: (i, 0)),
        out_shape=jax.ShapeDtypeStruct((BATCH, 128), jnp.float32),
    )(user_features, W1, b1.reshape(1, 32), W2, b2.reshape(1, EMBED_DIM))


def _sc_score(user_ids, item_ids, utT, itT, c_pad, tail_u, tail_i):
    mesh = plsc.VectorSubcoreMesh(core_axis_name="c", subcore_axis_name="s")

    @functools.partial(
        pl.kernel,
        mesh=mesh,
        out_type=jax.ShapeDtypeStruct((BATCH,), jnp.float32),
        scratch_types=[
            pltpu.VMEM((B_PER_W,), jnp.int32),
            pltpu.VMEM((B_PER_W,), jnp.int32),
            pltpu.VMEM((C_CHUNK, 128), jnp.float32),
            pltpu.VMEM((GRP, 16, 128), jnp.float32),
            pltpu.VMEM((GRP, 16, 128), jnp.float32),
            pltpu.VMEM((16, 64), jnp.float32),
            pltpu.VMEM((16, 64), jnp.float32),
            pltpu.VMEM((B_PER_W,), jnp.float32),
            pltpu.SemaphoreType.DMA,
            pltpu.SemaphoreType.DMA,
        ],
        compiler_params=pltpu.CompilerParams(
            use_tc_tiling_on_sc=True, needs_layout_passes=False),
    )
    def score_kernel(uid_hbm, iid_hbm, utT_hbm, itT_hbm, cpad_hbm,
                     tailu_hbm, taili_hbm, out_hbm,
                     idx_u, idx_i, crows, su, si, tu, ti, sbuf, sem, sem_c):
        wid = lax.axis_index("s") * NC + lax.axis_index("c")
        base = wid * B_PER_W
        pltpu.sync_copy(uid_hbm.at[pl.ds(base, B_PER_W)], idx_u)
        pltpu.sync_copy(iid_hbm.at[pl.ds(base, B_PER_W)], idx_i)
        pltpu.sync_copy(tailu_hbm, tu)
        pltpu.sync_copy(taili_hbm, ti)

        lane_iota = lax.iota(jnp.int32, 16)

        def extract(buf, lane):
            lanes = jnp.broadcast_to(lane, (16,))
            return plsc.load_gather(buf, [lane_iota, lanes])

        def chunk_body(cc, _):
            cbase = cc * C_CHUNK
            pltpu.sync_copy(
                cpad_hbm.at[pl.ds(base + cbase, C_CHUNK)], crows)

            def group_body(g, _):
                gbase = cbase + g * GRP
                ids_u = idx_u[pl.ds(gbase, 16)]
                ids_i = idx_i[pl.ds(gbase, 16)]
                for k in range(GRP):
                    u = ids_u[k]
                    i = ids_i[k]
                    au = pl.multiple_of(
                        jnp.minimum((u >> 7) << 7, TAIL_START - 128), 128)
                    ai = pl.multiple_of(
                        jnp.minimum((i >> 7) << 7, TAIL_START - 128), 128)
                    pltpu.async_copy(
                        utT_hbm.at[:, pl.ds(au, 128)], su.at[k], sem)
                    pltpu.async_copy(
                        itT_hbm.at[:, pl.ds(ai, 128)], si.at[k], sem)
                for k in range(GRP):
                    pltpu.make_async_copy(
                        utT_hbm.at[:, pl.ds(0, 128)], su.at[k], sem).wait()
                    pltpu.make_async_copy(
                        itT_hbm.at[:, pl.ds(0, 128)], si.at[k], sem).wait()
                acc = jnp.zeros((16,), jnp.float32)
                for k in range(GRP):
                    u = ids_u[k]
                    i = ids_i[k]
                    uvec = jnp.where(
                        u >= TAIL_START,
                        extract(tu, jnp.clip(u - TAIL_START, 0, 63)),
                        extract(su.at[k], u & 127))
                    ivec = jnp.where(
                        i >= TAIL_START,
                        extract(ti, jnp.clip(i - TAIL_START, 0, 63)),
                        extract(si.at[k], i & 127))
                    cvec = crows[g * GRP + k, pl.ds(0, 16)]
                    s = jnp.sum((uvec + cvec) * ivec, axis=0)
                    acc = jnp.where(lane_iota == k, s, acc)
                sbuf[pl.ds(gbase, GRP)] = acc
                return ()

            lax.fori_loop(0, C_CHUNK // GRP, group_body, (), unroll=False)
            return ()

        lax.fori_loop(0, B_PER_W // C_CHUNK, chunk_body, (), unroll=False)
        pltpu.sync_copy(sbuf, out_hbm.at[pl.ds(base, B_PER_W)])

    return score_kernel(user_ids, item_ids, utT, itT, c_pad, tail_u, tail_i)


def kernel(user_ids, item_ids, user_features, cf_user_table, cf_item_table,
           W1, b1, W2, b2):
    utT = cf_user_table.T
    itT = cf_item_table.T
    tail_u = lax.slice(utT, (0, TAIL_START), (EMBED_DIM, N_ROWS))
    tail_i = lax.slice(itT, (0, TAIL_START), (EMBED_DIM, N_ROWS))
    c_pad = _tc_mlp(user_features, W1, b1, W2, b2)
    return _sc_score(user_ids, item_ids, utT, itT, c_pad, tail_u, tail_i)
